# Initial kernel scaffold; baseline (speedup 1.0000x reference)
import jax
import jax.numpy as jnp
from jax.experimental import pallas as pl


def kernel(X, W, Wn1, bn1, Wa1, ba1, Wm1, bm1, Wm2, bm2, Wo1, bo1, Wo2, bo2):
    raise NotImplementedError("write your pallas kernel here")


# speedup vs baseline: 1.8250x; 1.8250x over previous
import jax
import jax.numpy as jnp
from jax.experimental import pallas as pl

_D = 64
_H = 24
_BB = 128
_IB = 4


def _contract(w, t):
    acc = w[:, 0][None, :, None] * t[:, 0, :][:, None, :]
    for l in range(1, _H):
        acc = acc + w[:, l][None, :, None] * t[:, l, :][:, None, :]
    return acc


def _core(xt_ref, wn1_ref, bn1_ref, wm1a_ref, wm1b_ref, bm1_ref, wm2_ref,
          bms_ref, at_ref, wo1h_ref, wo1g_ref, bo1_ref, wo2_ref, bo2_ref,
          out_ref):
    x = xt_ref[:]
    wn1 = wn1_ref[:]
    bn1 = bn1_ref[:]
    h = jnp.tanh(x[:, None, :] * wn1[None, :, :] + bn1[None, :, :])

    u = _contract(wm1a_ref[:], h)
    v = _contract(wm1b_ref[:], h) + bm1_ref[:][None, :, :]

    at = at_ref[:]
    pieces = []
    for ib in range(0, _D, _IB):
        vb = v[ib:ib + _IB]
        ab = at[ib:ib + _IB]
        p = jnp.maximum(u[None, :, :, :] + vb[:, None, :, :], 0.0)
        pieces.append((p * ab[:, :, None, None]).sum(axis=1))
    red = jnp.concatenate(pieces, axis=0)

    bms = bms_ref[:]
    agg = _contract(wm2_ref[:], red) + bms[:, :, None]

    o1 = jnp.maximum(_contract(wo1h_ref[:], h) + _contract(wo1g_ref[:], agg)
                     + bo1_ref[:][None, :, :], 0.0)

    wo2 = wo2_ref[:]
    out_ref[:] = (o1 * wo2[None, :, :]).sum(axis=1) + bo2_ref[0, 0]


def kernel(X, W, Wn1, bn1, Wa1, ba1, Wm1, bm1, Wm2, bm2, Wo1, bo1, Wo2, bo2):
    B, d = X.shape
    A = W * (1.0 - jnp.eye(d, dtype=W.dtype))
    At = A.T
    bms = At.sum(axis=1)[:, None] * bm2[None, :]
    Xt = X.T

    inputs = [
        Xt,
        Wn1,
        bn1[:, None],
        Wm1[:, :_H],
        Wm1[:, _H:],
        bm1[:, None],
        Wm2,
        bms,
        At,
        Wo1[:, :_H],
        Wo1[:, _H:],
        bo1[:, None],
        Wo2.T,
        bo2[:, None],
    ]

    full = lambda a: pl.BlockSpec(a.shape, lambda g: (0,) * a.ndim)
    in_specs = [pl.BlockSpec((d, _BB), lambda g: (0, g))]
    in_specs += [full(a) for a in inputs[1:]]

    out_t = pl.pallas_call(
        _core,
        grid=(B // _BB,),
        in_specs=in_specs,
        out_specs=pl.BlockSpec((d, _BB), lambda g: (0, g)),
        out_shape=jax.ShapeDtypeStruct((d, B), X.dtype),
    )(*inputs)
    return out_t.T


# speedup vs baseline: 2.7277x; 1.4946x over previous
import jax
import jax.numpy as jnp
from jax.experimental import pallas as pl

_D = 64
_H = 24
_BB = 128
_IB = 8
_LW = _H * _BB


def _core(xt_ref, wn1r_ref, bn1r_ref, wm1a_ref, wm1b_ref, bm1_ref,
          atbd_ref, wog2_ref, wo1h_ref, bo1i_ref, wo2_ref, bo2_ref,
          out_ref):
    x = xt_ref[:]
    xr = jnp.concatenate([x] * _H, axis=1)
    h2 = jnp.tanh(xr * wn1r_ref[:] + bn1r_ref[:])

    wa = wm1a_ref[:]
    wb = wm1b_ref[:]
    wh = wo1h_ref[:]
    bm1 = bm1_ref[:]
    u_p, v_p, hh_p = [], [], []
    for k in range(_H):
        au = av = ah = None
        for l in range(_H):
            hl = h2[:, l * _BB:(l + 1) * _BB]
            pu = hl * wa[k:k + 1, l:l + 1]
            pv = hl * wb[k:k + 1, l:l + 1]
            ph = hl * wh[k:k + 1, l:l + 1]
            au = pu if au is None else au + pu
            av = pv if av is None else av + pv
            ah = ph if ah is None else ah + ph
        u_p.append(au)
        v_p.append(av + bm1[k:k + 1, 0:1])
        hh_p.append(ah)
    u2 = jnp.concatenate(u_p, axis=1)
    v2 = jnp.concatenate(v_p, axis=1)
    hh2 = jnp.concatenate(hh_p, axis=1)

    atbd = atbd_ref[:]
    red_p = []
    for g in range(0, _D, _IB):
        t_parts = [jnp.maximum(u2 + v2[i:i + 1, :], 0.0)
                   for i in range(g, g + _IB)]
        t = jnp.concatenate(t_parts, axis=0)
        red_p.append(jnp.dot(atbd[g:g + _IB, :], t,
                             preferred_element_type=jnp.float32))
    red2 = jnp.concatenate(red_p, axis=0)

    wg = wog2_ref[:]
    o_p = []
    for k in range(_H):
        acc = None
        for l in range(_H):
            p = red2[:, l * _BB:(l + 1) * _BB] * wg[k:k + 1, l:l + 1]
            acc = p if acc is None else acc + p
        o_p.append(acc)
    og = jnp.concatenate(o_p, axis=1)

    o1 = jnp.maximum(hh2 + og + bo1i_ref[:], 0.0)
    wo2 = wo2_ref[:]
    out = None
    for k in range(_H):
        p = o1[:, k * _BB:(k + 1) * _BB] * wo2[0:1, k:k + 1]
        out = p if out is None else out + p
    out_ref[:] = out + bo2_ref[0:1, 0:1]


def kernel(X, W, Wn1, bn1, Wa1, ba1, Wm1, bm1, Wm2, bm2, Wo1, bo1, Wo2, bo2):
    B, d = X.shape
    f32 = jnp.float32
    A = W * (1.0 - jnp.eye(d, dtype=W.dtype))
    At = A.T
    s = At.sum(axis=1)
    bms = s[:, None] * bm2[None, :]
    bo1i = bms @ Wo1[:, _H:].T + bo1[None, :]
    bo1i_rep = jnp.repeat(bo1i, _BB, axis=1)
    wog2 = Wo1[:, _H:] @ Wm2
    oh = (jnp.arange(d)[:, None] % _IB ==
          jnp.arange(_IB)[None, :]).astype(f32)
    atbd = (oh[:, :, None] * At[:, None, :]).reshape(d, _IB * d)

    wn1r = jnp.repeat(Wn1[:, 0], _BB)[None, :]
    bn1r = jnp.repeat(bn1, _BB)[None, :]

    inputs = [
        X.T,
        wn1r,
        bn1r,
        Wm1[:, :_H],
        Wm1[:, _H:],
        bm1[:, None],
        atbd,
        wog2,
        Wo1[:, :_H],
        bo1i_rep,
        Wo2,
        bo2[:, None],
    ]

    full = lambda a: pl.BlockSpec(a.shape, lambda g: (0,) * a.ndim)
    in_specs = [pl.BlockSpec((d, _BB), lambda g: (0, g))]
    in_specs += [full(a) for a in inputs[1:]]

    out_t = pl.pallas_call(
        _core,
        grid=(B // _BB,),
        in_specs=in_specs,
        out_specs=pl.BlockSpec((d, _BB), lambda g: (0, g)),
        out_shape=jax.ShapeDtypeStruct((d, B), X.dtype),
    )(*inputs)
    return out_t.T


# speedup vs baseline: 3.5643x; 1.3067x over previous
import jax
import jax.numpy as jnp
from jax.experimental import pallas as pl

_D = 64
_H = 24
_BB = 128
_IB = 8
_LW = _H * _BB


def _core(xt_ref, wn1_ref, bn1_ref, wm1a_ref, wm1b_ref, bm1_ref,
          atbd_ref, wog2_ref, wo1h_ref, bo1it_ref, wo2_ref, bo2_ref,
          out_ref):
    x = xt_ref[:]
    xf = x.reshape(1, _D * _BB)
    h_t = jnp.tanh(wn1_ref[:] * xf + bn1_ref[:])

    u_t = jnp.dot(wm1a_ref[:], h_t, preferred_element_type=jnp.float32)
    v_t = jnp.dot(wm1b_ref[:], h_t, preferred_element_type=jnp.float32)
    v_t = v_t + bm1_ref[:]
    hh_t = jnp.dot(wo1h_ref[:], h_t, preferred_element_type=jnp.float32)

    u2 = jnp.transpose(u_t.reshape(_H, _D, _BB), (1, 0, 2)).reshape(_D, _LW)
    v2 = jnp.transpose(v_t.reshape(_H, _D, _BB), (1, 0, 2)).reshape(_D, _LW)

    atbd = atbd_ref[:]
    red_p = []
    for g in range(0, _D, _IB):
        t_parts = [jnp.maximum(u2 + v2[i:i + 1, :], 0.0)
                   for i in range(g, g + _IB)]
        t = jnp.concatenate(t_parts, axis=0)
        red_p.append(jnp.dot(atbd[g:g + _IB, :], t,
                             preferred_element_type=jnp.float32))
    red2 = jnp.concatenate(red_p, axis=0)

    red_t = jnp.transpose(red2.reshape(_D, _H, _BB), (1, 0, 2))
    red_t = red_t.reshape(_H, _D * _BB)

    o1 = jnp.maximum(hh_t + jnp.dot(wog2_ref[:], red_t,
                                    preferred_element_type=jnp.float32)
                     + bo1it_ref[:], 0.0)
    out = jnp.dot(wo2_ref[:], o1, preferred_element_type=jnp.float32)
    out = out + bo2_ref[0:1, 0:1]
    out_ref[:] = out.reshape(_D, _BB)


def kernel(X, W, Wn1, bn1, Wa1, ba1, Wm1, bm1, Wm2, bm2, Wo1, bo1, Wo2, bo2):
    B, d = X.shape
    f32 = jnp.float32
    A = W * (1.0 - jnp.eye(d, dtype=W.dtype))
    At = A.T
    s = At.sum(axis=1)
    bms = s[:, None] * bm2[None, :]
    bo1i = bms @ Wo1[:, _H:].T + bo1[None, :]
    bo1it = jnp.repeat(bo1i.T, _BB, axis=1)
    wog2 = Wo1[:, _H:] @ Wm2
    oh = (jnp.arange(d)[:, None] % _IB ==
          jnp.arange(_IB)[None, :]).astype(f32)
    atbd = (oh[:, :, None] * At[:, None, :]).reshape(d, _IB * d)

    inputs = [
        X.T,
        Wn1,
        bn1[:, None],
        Wm1[:, :_H],
        Wm1[:, _H:],
        bm1[:, None],
        atbd,
        wog2,
        Wo1[:, :_H],
        bo1it,
        Wo2,
        bo2[:, None],
    ]

    full = lambda a: pl.BlockSpec(a.shape, lambda g: (0,) * a.ndim)
    in_specs = [pl.BlockSpec((d, _BB), lambda g: (0, g))]
    in_specs += [full(a) for a in inputs[1:]]

    out_t = pl.pallas_call(
        _core,
        grid=(B // _BB,),
        in_specs=in_specs,
        out_specs=pl.BlockSpec((d, _BB), lambda g: (0, g)),
        out_shape=jax.ShapeDtypeStruct((d, B), X.dtype),
    )(*inputs)
    return out_t.T


# speedup vs baseline: 3.6588x; 1.0265x over previous
import jax
import jax.numpy as jnp
from jax.experimental import pallas as pl

_D = 64
_H = 24
_BB = 128
_IB = 8
_LW = _H * _BB
_BF = jnp.bfloat16


def _core(xf_ref, wn1_ref, bn1_ref, wm1a_ref, wm1b_ref, bm1_ref,
          atbd_ref, wog2_ref, wo1h_ref, bo1it_ref, wo2_ref, bo2_ref,
          out_ref):
    xf = xf_ref[:].reshape(1, _D * _BB)
    h_t = jnp.tanh(wn1_ref[:] * xf + bn1_ref[:])

    u_t = jnp.dot(wm1a_ref[:], h_t, preferred_element_type=jnp.float32)
    v_t = jnp.dot(wm1b_ref[:], h_t, preferred_element_type=jnp.float32)
    v_t = v_t + bm1_ref[:]
    hh_t = jnp.dot(wo1h_ref[:], h_t, preferred_element_type=jnp.float32)

    u2 = jnp.transpose(u_t.astype(_BF).reshape(_H, _D, _BB),
                       (1, 0, 2)).reshape(_D, _LW)
    v2 = jnp.transpose(v_t.astype(_BF).reshape(_H, _D, _BB),
                       (1, 0, 2)).reshape(_D, _LW)

    atbd = atbd_ref[:]
    red_p = []
    for g in range(0, _D, _IB):
        t_parts = [jnp.maximum(u2 + v2[i:i + 1, :], 0.0)
                   for i in range(g, g + _IB)]
        t = jnp.concatenate(t_parts, axis=0)
        red_p.append(jnp.dot(atbd[g:g + _IB, :], t,
                             preferred_element_type=jnp.float32))
    red2 = jnp.concatenate(red_p, axis=0)

    red_t = jnp.transpose(red2.reshape(_D, _H, _BB), (1, 0, 2))
    red_t = red_t.reshape(_H, _D * _BB)

    o1 = jnp.maximum(hh_t + jnp.dot(wog2_ref[:], red_t,
                                    preferred_element_type=jnp.float32)
                     + bo1it_ref[:], 0.0)
    out = jnp.dot(wo2_ref[:], o1, preferred_element_type=jnp.float32)
    out_ref[:] = (out + bo2_ref[0:1, 0:1]).reshape(1, 1, _D * _BB)


def kernel(X, W, Wn1, bn1, Wa1, ba1, Wm1, bm1, Wm2, bm2, Wo1, bo1, Wo2, bo2):
    B, d = X.shape
    f32 = jnp.float32
    nsteps = B // _BB
    A = W * (1.0 - jnp.eye(d, dtype=W.dtype))
    At = A.T
    s = At.sum(axis=1)
    bms = s[:, None] * bm2[None, :]
    bo1i = bms @ Wo1[:, _H:].T + bo1[None, :]
    bo1it = jnp.repeat(bo1i.T, _BB, axis=1)
    wog2 = Wo1[:, _H:] @ Wm2
    oh = (jnp.arange(d)[:, None] % _IB ==
          jnp.arange(_IB)[None, :]).astype(f32)
    atbd = (oh[:, :, None] * At[:, None, :]).reshape(d, _IB * d)

    xflat = X.T.reshape(d, nsteps, _BB).transpose(1, 0, 2).reshape(
        nsteps, 1, d * _BB)

    inputs = [
        xflat,
        Wn1,
        bn1[:, None],
        Wm1[:, :_H],
        Wm1[:, _H:],
        bm1[:, None],
        atbd.astype(_BF),
        wog2,
        Wo1[:, :_H],
        bo1it,
        Wo2,
        bo2[:, None],
    ]

    full = lambda a: pl.BlockSpec(a.shape, lambda g: (0,) * a.ndim)
    in_specs = [pl.BlockSpec((1, 1, d * _BB), lambda g: (g, 0, 0))]
    in_specs += [full(a) for a in inputs[1:]]

    out_flat = pl.pallas_call(
        _core,
        grid=(nsteps,),
        in_specs=in_specs,
        out_specs=pl.BlockSpec((1, 1, d * _BB), lambda g: (g, 0, 0)),
        out_shape=jax.ShapeDtypeStruct((nsteps, 1, d * _BB), X.dtype),
    )(*inputs)
    return out_flat.reshape(nsteps, d, _BB).transpose(0, 2, 1).reshape(B, d)
